# Initial kernel scaffold; baseline (speedup 1.0000x reference)
import functools

import jax
import jax.numpy as jnp
from jax import lax
from jax.experimental import pallas as pl
from jax.experimental.pallas import tpu as pltpu
from jax.experimental.pallas import tpu_sc as plsc

N_NODES = 10000
N_EDGES = 320000
IN_SIZE = 128
HIDDEN = 256
OUT_SIZE = 128
NUM_PROP_LAYERS = 10
ALPHA = 0.1

D = OUT_SIZE
NUM_TILES = 16
EDGES_PER_TILE = N_EDGES // NUM_TILES
G = 80
MACRO = 4000
GATHERS_PER_MACRO = MACRO // G
MACROS_PER_TILE = EDGES_PER_TILE // MACRO
ROWS_PER_TILE = N_NODES // NUM_TILES
RB = 125
FLUSHES = ROWS_PER_TILE // RB


def _mlp_body(x_ref, w1_ref, b1_ref, w2_ref, b2_ref, h_ref, ah_ref):
    h = jnp.maximum(
        jax.lax.dot(x_ref[...], w1_ref[...],
                    preferred_element_type=jnp.float32,
                    precision=jax.lax.Precision.HIGHEST) + b1_ref[...],
        0.0,
    )
    o = jax.lax.dot(h, w2_ref[...],
                    preferred_element_type=jnp.float32,
                    precision=jax.lax.Precision.HIGHEST) + b2_ref[...]
    h_ref[...] = o
    ah_ref[...] = o * ALPHA


def _mlp(H, W1, b1, W2, b2):
    BM = 2000
    grid = (N_NODES // BM,)
    return pl.pallas_call(
        _mlp_body,
        grid=grid,
        in_specs=[
            pl.BlockSpec((BM, IN_SIZE), lambda i: (i, 0)),
            pl.BlockSpec((IN_SIZE, HIDDEN), lambda i: (0, 0)),
            pl.BlockSpec((1, HIDDEN), lambda i: (0, 0)),
            pl.BlockSpec((HIDDEN, OUT_SIZE), lambda i: (0, 0)),
            pl.BlockSpec((1, OUT_SIZE), lambda i: (0, 0)),
        ],
        out_specs=[
            pl.BlockSpec((BM, OUT_SIZE), lambda i: (i, 0)),
            pl.BlockSpec((BM, OUT_SIZE), lambda i: (i, 0)),
        ],
        out_shape=[
            jax.ShapeDtypeStruct((N_NODES, OUT_SIZE), jnp.float32),
            jax.ShapeDtypeStruct((N_NODES, OUT_SIZE), jnp.float32),
        ],
    )(H, W1, b1.reshape(1, HIDDEN), W2, b2.reshape(1, OUT_SIZE))


def _prop_body(h0, ah, src, dst2d, aval, out,
               srcB, avalB, dstB, rows, fb, acc, sem):
    wid = lax.axis_index("s")
    tile_e0 = wid * EDGES_PER_TILE
    tile_r0 = wid * ROWS_PER_TILE

    def init_chunk(b, carry):
        r0 = tile_r0 + b * RB
        pltpu.sync_copy(h0.at[pl.ds(r0, RB)], fb)
        pltpu.sync_copy(fb, out.at[pl.ds(r0, RB)])
        pltpu.sync_copy(ah.at[pl.ds(r0, RB)], fb)
        pltpu.sync_copy(fb, acc.at[pl.ds(r0, RB)])
        return carry
    lax.fori_loop(0, FLUSHES, init_chunk, 0)
    plsc.subcore_barrier()

    def step(s, carry0):
        def macro(m, carry):
            e0 = tile_e0 + m * MACRO
            pltpu.sync_copy(src.at[pl.ds(e0, MACRO)], srcB)
            pltpu.sync_copy(aval.at[pl.ds(e0, MACRO)], avalB)
            pltpu.sync_copy(dst2d.at[pl.ds(e0 // G, GATHERS_PER_MACRO)], dstB)

            def chunk(j, c2):
                pltpu.async_copy(
                    out.at[srcB.at[pl.ds(j * G, G)]], rows, sem).wait()

                def scale_row(r, c3):
                    e = j * G + r
                    sc = plsc.load_gather(
                        avalB, [jnp.full((16,), e, jnp.int32)])
                    for i in range(D // 16):
                        sl = pl.ds(i * 16, 16)
                        rows[r, sl] = rows[r, sl] * sc
                    return c3
                lax.fori_loop(0, G, scale_row, 0)
                pltpu.sync_copy(rows, acc.at[dstB.at[j]], add=True)
                return c2
            lax.fori_loop(0, GATHERS_PER_MACRO, chunk, 0)
            return carry
        lax.fori_loop(0, MACROS_PER_TILE, macro, 0)
        plsc.subcore_barrier()

        def flush_chunk(b, carry):
            r0 = tile_r0 + b * RB
            pltpu.sync_copy(acc.at[pl.ds(r0, RB)], fb)
            pltpu.sync_copy(fb, out.at[pl.ds(r0, RB)])
            pltpu.sync_copy(ah.at[pl.ds(r0, RB)], fb)
            pltpu.sync_copy(fb, acc.at[pl.ds(r0, RB)])
            return carry
        lax.fori_loop(0, FLUSHES, flush_chunk, 0)
        plsc.subcore_barrier()
        return carry0

    lax.fori_loop(0, NUM_PROP_LAYERS, step, 0)


def _propagate(h_local, alpha_h, src, dst2d, aval):
    mesh = plsc.VectorSubcoreMesh(
        core_axis_name="c", subcore_axis_name="s", num_cores=1)
    f = pl.kernel(
        _prop_body,
        out_type=jax.ShapeDtypeStruct((N_NODES, D), jnp.float32),
        mesh=mesh,
        scratch_types=[
            pltpu.VMEM((MACRO,), jnp.int32),
            pltpu.VMEM((MACRO,), jnp.float32),
            pltpu.VMEM((GATHERS_PER_MACRO, G), jnp.int32),
            pltpu.VMEM((G, D), jnp.float32),
            pltpu.VMEM((RB, D), jnp.float32),
            pltpu.MemorySpace.VMEM_SHARED((N_NODES, D), jnp.float32),
            pltpu.SemaphoreType.DMA,
        ],
    )
    return f(h_local, alpha_h, src, dst2d, aval)


def kernel(H, A_val, edge_index, W1, b1, W2, b2):
    h_local, alpha_h = _mlp(H, W1, b1, W2, b2)
    src = edge_index[0].astype(jnp.int32)
    dst = edge_index[1].astype(jnp.int32)
    dst2d = dst.reshape(N_EDGES // G, G)
    return _propagate(h_local, alpha_h, src, dst2d, A_val)


# speedup vs baseline: 2.8586x; 2.8586x over previous
import functools

import jax
import jax.numpy as jnp
from jax import lax
from jax.experimental import pallas as pl
from jax.experimental.pallas import tpu as pltpu
from jax.experimental.pallas import tpu_sc as plsc

N_NODES = 10000
N_EDGES = 320000
IN_SIZE = 128
HIDDEN = 256
OUT_SIZE = 128
NUM_PROP_LAYERS = 10
ALPHA = 0.1

D = OUT_SIZE
NUM_TILES = 16
EDGES_PER_TILE = N_EDGES // NUM_TILES
G = 80
CHUNKS_PER_TILE = EDGES_PER_TILE // G
RB = 80
N_ROW_CHUNKS = N_NODES // RB


def _mlp_body(x_ref, w1_ref, b1_ref, w2_ref, b2_ref, h_ref, ah_ref):
    h = jnp.maximum(
        jax.lax.dot(x_ref[...], w1_ref[...],
                    preferred_element_type=jnp.float32,
                    precision=jax.lax.Precision.HIGHEST) + b1_ref[...],
        0.0,
    )
    o = jax.lax.dot(h, w2_ref[...],
                    preferred_element_type=jnp.float32,
                    precision=jax.lax.Precision.HIGHEST) + b2_ref[...]
    h_ref[...] = o
    ah_ref[...] = o * ALPHA


def _mlp(H, W1, b1, W2, b2):
    BM = 2000
    grid = (N_NODES // BM,)
    return pl.pallas_call(
        _mlp_body,
        grid=grid,
        in_specs=[
            pl.BlockSpec((BM, IN_SIZE), lambda i: (i, 0)),
            pl.BlockSpec((IN_SIZE, HIDDEN), lambda i: (0, 0)),
            pl.BlockSpec((1, HIDDEN), lambda i: (0, 0)),
            pl.BlockSpec((HIDDEN, OUT_SIZE), lambda i: (0, 0)),
            pl.BlockSpec((1, OUT_SIZE), lambda i: (0, 0)),
        ],
        out_specs=[
            pl.BlockSpec((BM, OUT_SIZE), lambda i: (i, 0)),
            pl.BlockSpec((BM, OUT_SIZE), lambda i: (i, 0)),
        ],
        out_shape=[
            jax.ShapeDtypeStruct((N_NODES, OUT_SIZE), jnp.float32),
            jax.ShapeDtypeStruct((N_NODES, OUT_SIZE), jnp.float32),
        ],
    )(H, W1, b1.reshape(1, HIDDEN), W2, b2.reshape(1, OUT_SIZE))


MACRO = 2000
CHUNKS_PER_MACRO = MACRO // G
MACROS_PER_TILE = EDGES_PER_TILE // MACRO


def _prop_body(h0, ah, src, dst3, aval, out,
               srcB, avalB, dstB, rows, acc, sem):
    wid = lax.axis_index("s")
    tile_e0 = wid * EDGES_PER_TILE

    pltpu.sync_copy(dst3.at[wid], dstB)

    def my_chunk(k):
        return (wid + k * NUM_TILES) * RB

    nck = (N_ROW_CHUNKS - 1 - wid) // NUM_TILES + 1

    def init_chunk(k, carry):
        r0 = my_chunk(k)
        pltpu.sync_copy(h0.at[pl.ds(r0, RB)], rows)
        pltpu.sync_copy(rows, out.at[pl.ds(r0, RB)])
        pltpu.sync_copy(ah.at[pl.ds(r0, RB)], rows)
        pltpu.sync_copy(rows, acc.at[pl.ds(r0, RB)])
        return carry
    lax.fori_loop(0, nck, init_chunk, 0)
    plsc.subcore_barrier()

    def step(s, carry0):
        def macro(m, c1):
            e0 = tile_e0 + m * MACRO
            pltpu.sync_copy(src.at[pl.ds(e0, MACRO)], srcB)
            pltpu.sync_copy(aval.at[pl.ds(e0, MACRO)], avalB)

            def chunk(j, c2):
                pltpu.async_copy(
                    out.at[srcB.at[pl.ds(j * G, G)]], rows, sem).wait()

                def scale_group(b, c3):
                    av16 = avalB[pl.ds(j * G + b * 16, 16)]
                    for r16 in range(16):
                        sc = lax.gather(
                            av16,
                            jnp.full((16, 1), r16, jnp.int32),
                            lax.GatherDimensionNumbers(
                                offset_dims=(),
                                collapsed_slice_dims=(0,),
                                start_index_map=(0,)),
                            (1,),
                            mode=lax.GatherScatterMode.PROMISE_IN_BOUNDS)
                        r = b * 16 + r16
                        for i in range(D // 16):
                            sl = pl.ds(i * 16, 16)
                            rows[r, sl] = rows[r, sl] * sc
                    return c3
                lax.fori_loop(0, G // 16, scale_group, 0)
                pltpu.sync_copy(rows, acc.at[dstB.at[m * CHUNKS_PER_MACRO + j]],
                                add=True)
                return c2
            lax.fori_loop(0, CHUNKS_PER_MACRO, chunk, 0)
            return c1
        lax.fori_loop(0, MACROS_PER_TILE, macro, 0)
        plsc.subcore_barrier()

        def flush_chunk(k, carry):
            r0 = my_chunk(k)
            pltpu.sync_copy(acc.at[pl.ds(r0, RB)], rows)
            pltpu.sync_copy(rows, out.at[pl.ds(r0, RB)])
            pltpu.sync_copy(ah.at[pl.ds(r0, RB)], rows)
            pltpu.sync_copy(rows, acc.at[pl.ds(r0, RB)])
            return carry
        lax.fori_loop(0, nck, flush_chunk, 0)
        plsc.subcore_barrier()
        return carry0

    lax.fori_loop(0, NUM_PROP_LAYERS, step, 0)


def _propagate(h_local, alpha_h, src, dst3, aval):
    mesh = plsc.VectorSubcoreMesh(
        core_axis_name="c", subcore_axis_name="s", num_cores=1)
    f = pl.kernel(
        _prop_body,
        out_type=jax.ShapeDtypeStruct((N_NODES, D), jnp.float32),
        mesh=mesh,
        scratch_types=[
            pltpu.VMEM((MACRO,), jnp.int32),
            pltpu.VMEM((MACRO,), jnp.float32),
            pltpu.VMEM((CHUNKS_PER_TILE, G), jnp.int32),
            pltpu.VMEM((G, D), jnp.float32),
            pltpu.VMEM_SHARED((N_NODES, D), jnp.float32),
            pltpu.SemaphoreType.DMA,
        ],
    )
    return f(h_local, alpha_h, src, dst3, aval)


def kernel(H, A_val, edge_index, W1, b1, W2, b2):
    h_local, alpha_h = _mlp(H, W1, b1, W2, b2)
    src = edge_index[0].astype(jnp.int32)
    dst = edge_index[1].astype(jnp.int32)
    dst3 = dst.reshape(NUM_TILES, CHUNKS_PER_TILE, G)
    return _propagate(h_local, alpha_h, src, dst3, A_val)


# speedup vs baseline: 4.5568x; 1.5941x over previous
import functools

import jax
import jax.numpy as jnp
from jax import lax
from jax.experimental import pallas as pl
from jax.experimental.pallas import tpu as pltpu
from jax.experimental.pallas import tpu_sc as plsc

N_NODES = 10000
N_EDGES = 320000
IN_SIZE = 128
HIDDEN = 256
OUT_SIZE = 128
NUM_PROP_LAYERS = 10
ALPHA = 0.1

D = OUT_SIZE
NUM_TILES = 16
EDGES_PER_TILE = N_EDGES // NUM_TILES
G = 80
CHUNKS_PER_TILE = EDGES_PER_TILE // G
RB = 80
N_ROW_CHUNKS = N_NODES // RB


def _mlp_body(x_ref, w1_ref, b1_ref, w2_ref, b2_ref, h_ref, ah_ref):
    h = jnp.maximum(
        jax.lax.dot(x_ref[...], w1_ref[...],
                    preferred_element_type=jnp.float32,
                    precision=jax.lax.Precision.HIGHEST) + b1_ref[...],
        0.0,
    )
    o = jax.lax.dot(h, w2_ref[...],
                    preferred_element_type=jnp.float32,
                    precision=jax.lax.Precision.HIGHEST) + b2_ref[...]
    h_ref[...] = o
    ah_ref[...] = o * ALPHA


def _mlp(H, W1, b1, W2, b2):
    BM = 2000
    grid = (N_NODES // BM,)
    return pl.pallas_call(
        _mlp_body,
        grid=grid,
        in_specs=[
            pl.BlockSpec((BM, IN_SIZE), lambda i: (i, 0)),
            pl.BlockSpec((IN_SIZE, HIDDEN), lambda i: (0, 0)),
            pl.BlockSpec((1, HIDDEN), lambda i: (0, 0)),
            pl.BlockSpec((HIDDEN, OUT_SIZE), lambda i: (0, 0)),
            pl.BlockSpec((1, OUT_SIZE), lambda i: (0, 0)),
        ],
        out_specs=[
            pl.BlockSpec((BM, OUT_SIZE), lambda i: (i, 0)),
            pl.BlockSpec((BM, OUT_SIZE), lambda i: (i, 0)),
        ],
        out_shape=[
            jax.ShapeDtypeStruct((N_NODES, OUT_SIZE), jnp.float32),
            jax.ShapeDtypeStruct((N_NODES, OUT_SIZE), jnp.float32),
        ],
    )(H, W1, b1.reshape(1, HIDDEN), W2, b2.reshape(1, OUT_SIZE))


MACRO = 4000
CHUNKS_PER_MACRO = MACRO // G
MACROS_PER_TILE = EDGES_PER_TILE // MACRO
PAIRS_PER_MACRO = CHUNKS_PER_MACRO // 2


def _scale_rows(rows, avalB, base_e):
    def scale_group(b, c3):
        av16 = avalB[pl.ds(base_e + b * 16, 16)]
        for r16 in range(16):
            sc = lax.gather(
                av16,
                jnp.full((16, 1), r16, jnp.int32),
                lax.GatherDimensionNumbers(
                    offset_dims=(),
                    collapsed_slice_dims=(0,),
                    start_index_map=(0,)),
                (1,),
                mode=lax.GatherScatterMode.PROMISE_IN_BOUNDS)
            r = b * 16 + r16
            for i in range(D // 16):
                sl = pl.ds(i * 16, 16)
                rows[r, sl] = rows[r, sl] * sc
        return c3
    lax.fori_loop(0, G // 16, scale_group, 0)


def _prop_body(h0, ah, src, dst4, aval, out,
               srcB, avalB, dstB, rows0, rows1, acc,
               gsem0, gsem1, ssem0, ssem1):
    wid = lax.axis_index("s")
    tile_e0 = wid * EDGES_PER_TILE

    def my_chunk(k):
        return (wid + k * NUM_TILES) * RB

    nck = (N_ROW_CHUNKS - 1 - wid) // NUM_TILES + 1

    def init_chunk(k, carry):
        r0 = my_chunk(k)
        pltpu.sync_copy(h0.at[pl.ds(r0, RB)], rows0)
        pltpu.sync_copy(rows0, out.at[pl.ds(r0, RB)])
        pltpu.sync_copy(ah.at[pl.ds(r0, RB)], rows0)
        pltpu.sync_copy(rows0, acc.at[pl.ds(r0, RB)])
        return carry
    lax.fori_loop(0, nck, init_chunk, 0)
    plsc.subcore_barrier()

    def gather_start(c, buf, sem):
        pltpu.async_copy(out.at[srcB.at[pl.ds(c * G, G)]], buf, sem)

    def gather_wait(c, buf, sem):
        pltpu.make_async_copy(out.at[srcB.at[pl.ds(c * G, G)]], buf,
                              sem).wait()

    def scatter_start(c, buf, sem):
        pltpu.async_copy(buf, acc.at[dstB.at[c]], sem, add=True)

    def scatter_wait(c, buf, sem):
        pltpu.make_async_copy(buf, acc.at[dstB.at[c]], sem).wait()

    def step(s, carry0):
        def macro(m, c1):
            e0 = tile_e0 + m * MACRO
            pltpu.sync_copy(src.at[pl.ds(e0, MACRO)], srcB)
            pltpu.sync_copy(aval.at[pl.ds(e0, MACRO)], avalB)
            pltpu.sync_copy(dst4.at[wid, m], dstB)

            gather_start(0, rows0, gsem0)

            def pair(p, c2):
                c_a = 2 * p
                c_b = 2 * p + 1

                @pl.when(p > 0)
                def _():
                    scatter_wait(c_b - 2, rows1, ssem1)
                gather_start(c_b, rows1, gsem1)

                gather_wait(c_a, rows0, gsem0)
                _scale_rows(rows0, avalB, c_a * G)
                scatter_start(c_a, rows0, ssem0)

                gather_wait(c_b, rows1, gsem1)
                _scale_rows(rows1, avalB, c_b * G)
                scatter_start(c_b, rows1, ssem1)

                scatter_wait(c_a, rows0, ssem0)

                @pl.when(p < PAIRS_PER_MACRO - 1)
                def _():
                    gather_start(c_a + 2, rows0, gsem0)
                return c2
            lax.fori_loop(0, PAIRS_PER_MACRO, pair, 0)
            scatter_wait(CHUNKS_PER_MACRO - 1, rows1, ssem1)
            return c1
        lax.fori_loop(0, MACROS_PER_TILE, macro, 0)
        plsc.subcore_barrier()

        def flush_chunk(k, carry):
            r0 = my_chunk(k)
            pltpu.sync_copy(acc.at[pl.ds(r0, RB)], rows0)
            pltpu.sync_copy(rows0, out.at[pl.ds(r0, RB)])
            pltpu.sync_copy(ah.at[pl.ds(r0, RB)], rows0)
            pltpu.sync_copy(rows0, acc.at[pl.ds(r0, RB)])
            return carry
        lax.fori_loop(0, nck, flush_chunk, 0)
        plsc.subcore_barrier()
        return carry0

    lax.fori_loop(0, NUM_PROP_LAYERS, step, 0)


def _propagate(h_local, alpha_h, src, dst4, aval):
    mesh = plsc.VectorSubcoreMesh(
        core_axis_name="c", subcore_axis_name="s", num_cores=1)
    f = pl.kernel(
        _prop_body,
        out_type=jax.ShapeDtypeStruct((N_NODES, D), jnp.float32),
        mesh=mesh,
        scratch_types=[
            pltpu.VMEM((MACRO,), jnp.int32),
            pltpu.VMEM((MACRO,), jnp.float32),
            pltpu.VMEM((CHUNKS_PER_MACRO, G), jnp.int32),
            pltpu.VMEM((G, D), jnp.float32),
            pltpu.VMEM((G, D), jnp.float32),
            pltpu.VMEM_SHARED((N_NODES, D), jnp.float32),
            pltpu.SemaphoreType.DMA,
            pltpu.SemaphoreType.DMA,
            pltpu.SemaphoreType.DMA,
            pltpu.SemaphoreType.DMA,
        ],
    )
    return f(h_local, alpha_h, src, dst4, aval)


def kernel(H, A_val, edge_index, W1, b1, W2, b2):
    h_local, alpha_h = _mlp(H, W1, b1, W2, b2)
    src = edge_index[0].astype(jnp.int32)
    dst = edge_index[1].astype(jnp.int32)
    dst4 = dst.reshape(NUM_TILES, MACROS_PER_TILE, CHUNKS_PER_MACRO, G)
    return _propagate(h_local, alpha_h, src, dst4, A_val)
